# baseline (speedup 1.0000x reference)
import functools
import jax
import jax.numpy as jnp
from jax import lax
from jax.experimental import pallas as pl
from jax.experimental.pallas import tpu as pltpu
from jax.experimental.pallas import tpu_sc as plsc

BATCH = 4096
N_CLUSTERS = 8192
NC = 2
NS = 16
L = 16
NW = NC * NS
B_W = BATCH // NW
PG = 8
NV = N_CLUSTERS // L

_mesh = plsc.VectorSubcoreMesh(core_axis_name="c", subcore_axis_name="s")


_DNUMS = lax.GatherDimensionNumbers(
    offset_dims=(), collapsed_slice_dims=(0,), start_index_map=(0,))


def _bcast_lane(vec, lane):
    idx = jnp.full((L, 1), lane, jnp.int32)
    return lax.gather(vec, idx, _DNUMS, (1,),
                      mode=lax.GatherScatterMode.PROMISE_IN_BOUNDS)


def _lane_perm(vec, perm):
    idx = perm.reshape(L, 1)
    return lax.gather(vec, idx, _DNUMS, (1,),
                      mode=lax.GatherScatterMode.PROMISE_IN_BOUNDS)


def _allmin(vec):
    lanes = lax.broadcasted_iota(jnp.int32, (L,), 0)
    for s in (8, 4, 2, 1):
        vec = jnp.minimum(vec, _lane_perm(vec, lanes ^ s))
    return vec


@functools.partial(
    pl.kernel,
    mesh=_mesh,
    out_type=jax.ShapeDtypeStruct((BATCH,), jnp.int32),
    scratch_types=[
        pltpu.VMEM((B_W,), jnp.float32),
        pltpu.VMEM((B_W,), jnp.float32),
        pltpu.VMEM((N_CLUSTERS,), jnp.float32),
        pltpu.VMEM((N_CLUSTERS,), jnp.float32),
        pltpu.VMEM((B_W,), jnp.int32),
    ],
)
def _sc_assign(x0_hbm, x1_hbm, c0_hbm, c1_hbm, out_hbm,
               x0_v, x1_v, c0_v, c1_v, o_v):
    wid = lax.axis_index("s") * NC + lax.axis_index("c")
    base = wid * B_W
    pltpu.sync_copy(x0_hbm.at[pl.ds(base, B_W)], x0_v)
    pltpu.sync_copy(x1_hbm.at[pl.ds(base, B_W)], x1_v)
    pltpu.sync_copy(c0_hbm, c0_v)
    pltpu.sync_copy(c1_hbm, c1_v)

    lanes = lax.broadcasted_iota(jnp.int32, (L,), 0)

    def group16(g, _):
        x0c = x0_v[pl.ds(g * L, L)]
        x1c = x1_v[pl.ds(g * L, L)]
        res = jnp.zeros((L,), jnp.int32)

        for h in range(L // PG):
            x0b = [_bcast_lane(x0c, h * PG + p) for p in range(PG)]
            x1b = [_bcast_lane(x1c, h * PG + p) for p in range(PG)]

            def body(v, carry, x0b=x0b, x1b=x1b):
                bestv = list(carry[:PG])
                bidx = list(carry[PG:])
                c0c = c0_v[pl.ds(v * L, L)]
                c1c = c1_v[pl.ds(v * L, L)]
                for p in range(PG):
                    d0 = x0b[p] - c0c
                    d1 = x1b[p] - c1c
                    dist = d0 * d0 + d1 * d1
                    mask = dist < bestv[p]
                    bestv[p] = jnp.where(mask, dist, bestv[p])
                    bidx[p] = jnp.where(mask, v, bidx[p])
                return tuple(bestv) + tuple(bidx)

            init = tuple(jnp.full((L,), jnp.inf, jnp.float32)
                         for _ in range(PG)) + \
                   tuple(jnp.zeros((L,), jnp.int32) for _ in range(PG))
            carry = lax.fori_loop(0, NV, body, init, unroll=4)
            bestv = carry[:PG]
            bidx = carry[PG:]

            for p in range(PG):
                m = _allmin(bestv[p])
                cand = jnp.where(bestv[p] == m, bidx[p] * L + lanes,
                                 N_CLUSTERS)
                idxv = _allmin(cand)
                res = jnp.where(lanes == h * PG + p, idxv, res)

        o_v[pl.ds(g * L, L)] = res
        return 0

    lax.fori_loop(0, B_W // L, group16, 0)
    pltpu.sync_copy(o_v, out_hbm.at[pl.ds(base, B_W)])


def kernel(inputs, cluster_centers):
    x0 = inputs[:, 0]
    x1 = inputs[:, 1]
    c0 = cluster_centers[:, 0]
    c1 = cluster_centers[:, 1]
    return _sc_assign(x0, x1, c0, c1)


# speedup vs baseline: 3.6613x; 3.6613x over previous
import jax
import jax.numpy as jnp
from jax.experimental import pallas as pl
from jax.experimental.pallas import tpu as pltpu

BATCH = 4096
N_CLUSTERS = 8192
R = 1024
RG = 128
CK = 128


def _assign_kernel(x_ref, c_ref, out_ref):
    n_chunks = N_CLUSTERS // CK
    lanef = jax.lax.broadcasted_iota(jnp.int32, (RG, CK), 1).astype(jnp.float32)

    for g in range(R // RG):
        x0 = x_ref[pl.ds(g * RG, RG), 0:1]
        x1 = x_ref[pl.ds(g * RG, RG), 1:2]
        x0b = jnp.broadcast_to(x0, (RG, CK))
        x1b = jnp.broadcast_to(x1, (RG, CK))

        def body(t, carry, x0b=x0b, x1b=x1b):
            bestv, bidxf = carry
            c0 = c_ref[0:1, pl.ds(t * CK, CK)]
            c1 = c_ref[1:2, pl.ds(t * CK, CK)]
            d0 = x0b - c0
            d1 = x1b - c1
            dist = d0 * d0 + d1 * d1
            mask = dist < bestv
            bestv = jnp.where(mask, dist, bestv)
            bidxf = jnp.where(mask, t.astype(jnp.float32), bidxf)
            return bestv, bidxf

        bestv0 = jnp.full((RG, CK), jnp.inf, dtype=jnp.float32)
        bidxf0 = jnp.zeros((RG, CK), dtype=jnp.float32)
        bestv, bidxf = jax.lax.fori_loop(0, n_chunks, body, (bestv0, bidxf0),
                                         unroll=16)

        m = jnp.min(bestv, axis=-1, keepdims=True)
        cand = jnp.where(bestv == m, bidxf * CK + lanef, float(N_CLUSTERS))
        idxf = jnp.min(cand, axis=-1, keepdims=True)
        out_ref[pl.ds(g * RG, RG), :] = idxf.astype(jnp.int32)


def kernel(inputs, cluster_centers):
    centers_t = cluster_centers.T
    grid = (BATCH // R,)
    out2d = pl.pallas_call(
        _assign_kernel,
        grid=grid,
        in_specs=[
            pl.BlockSpec((R, 2), lambda i: (i, 0)),
            pl.BlockSpec((2, N_CLUSTERS), lambda i: (0, 0)),
        ],
        out_specs=pl.BlockSpec((R, 1), lambda i: (i, 0)),
        out_shape=jax.ShapeDtypeStruct((BATCH, 1), jnp.int32),
        compiler_params=pltpu.CompilerParams(
            dimension_semantics=("parallel",),
        ),
    )(inputs, centers_t)
    return out2d.reshape(BATCH)


# speedup vs baseline: 3.7149x; 1.0146x over previous
import jax
import jax.numpy as jnp
from jax.experimental import pallas as pl
from jax.experimental.pallas import tpu as pltpu

BATCH = 4096
N_CLUSTERS = 8192
R = 1024
RG = 128
CK = 128


def _assign_kernel(x_ref, c_ref, out_ref):
    n_chunks = N_CLUSTERS // CK
    lanef = jax.lax.broadcasted_iota(jnp.int32, (RG, CK), 1).astype(jnp.float32)

    for g in range(R // RG):
        x0 = x_ref[pl.ds(g * RG, RG), 0:1]
        x1 = x_ref[pl.ds(g * RG, RG), 1:2]
        x0b = jnp.broadcast_to(x0, (RG, CK))
        x1b = jnp.broadcast_to(x1, (RG, CK))

        def body(t, carry, x0b=x0b, x1b=x1b):
            bestv, bidxf = carry
            c0 = c_ref[0:1, pl.ds(t * CK, CK)]
            c1 = c_ref[1:2, pl.ds(t * CK, CK)]
            d0 = x0b - c0
            d1 = x1b - c1
            dist = d0 * d0 + d1 * d1
            mask = dist < bestv
            bestv = jnp.where(mask, dist, bestv)
            bidxf = jnp.where(mask, t.astype(jnp.float32), bidxf)
            return bestv, bidxf

        bestv0 = jnp.full((RG, CK), jnp.inf, dtype=jnp.float32)
        bidxf0 = jnp.zeros((RG, CK), dtype=jnp.float32)
        bestv, bidxf = jax.lax.fori_loop(0, n_chunks, body, (bestv0, bidxf0),
                                         unroll=32)

        m = jnp.min(bestv, axis=-1, keepdims=True)
        cand = jnp.where(bestv == m, bidxf * CK + lanef, float(N_CLUSTERS))
        idxf = jnp.min(cand, axis=-1, keepdims=True)
        out_ref[pl.ds(g * RG, RG), :] = idxf.astype(jnp.int32)


def kernel(inputs, cluster_centers):
    centers_t = cluster_centers.T
    grid = (BATCH // R,)
    out2d = pl.pallas_call(
        _assign_kernel,
        grid=grid,
        in_specs=[
            pl.BlockSpec((R, 2), lambda i: (i, 0)),
            pl.BlockSpec((2, N_CLUSTERS), lambda i: (0, 0)),
        ],
        out_specs=pl.BlockSpec((R, 1), lambda i: (i, 0)),
        out_shape=jax.ShapeDtypeStruct((BATCH, 1), jnp.int32),
        compiler_params=pltpu.CompilerParams(
            dimension_semantics=("parallel",),
        ),
    )(inputs, centers_t)
    return out2d.reshape(BATCH)
